# baseline (speedup 1.0000x reference)
import functools

import jax
import jax.numpy as jnp
from jax import lax
from jax.experimental import pallas as pl
from jax.experimental.pallas import tpu as pltpu
from jax.experimental.pallas import tpu_sc as plsc

DIM = 2048
SEQ = 8192
SCALE = DIM ** (-0.5)

NC = 2
NS = 16
NW = NC * NS

TOTAL = SEQ * DIM
PER_W = TOTAL // NW
CHUNK = 32_768
NCHUNK = PER_W // CHUNK
LANES = 16
UNROLL = 8

_mesh = plsc.VectorSubcoreMesh(core_axis_name="c", subcore_axis_name="s")


@functools.partial(
    pl.kernel,
    out_type=jax.ShapeDtypeStruct((TOTAL,), jnp.float32),
    mesh=_mesh,
    scratch_types=[
        pltpu.VMEM((CHUNK,), jnp.float32),
        pltpu.SemaphoreType.DMA,
    ],
)
def _scale_copy_sc(emb_hbm, out_hbm, buf, sem):
    wid = lax.axis_index("s") * NC + lax.axis_index("c")
    base = pl.multiple_of(wid * PER_W, CHUNK)

    def compute(i, _):
        off = i * (LANES * UNROLL)
        for u in range(UNROLL):
            o = off + u * LANES
            buf[pl.ds(o, LANES)] = buf[pl.ds(o, LANES)] * SCALE
        return 0

    for k in range(NCHUNK):
        off = base + k * CHUNK
        pltpu.async_copy(emb_hbm.at[pl.ds(off, CHUNK)], buf, sem).wait()
        lax.fori_loop(0, CHUNK // (LANES * UNROLL), compute, 0)
        pltpu.async_copy(buf, out_hbm.at[pl.ds(off, CHUNK)], sem).wait()


def kernel(x, emb):
    del x
    flat = _scale_copy_sc(emb.reshape(TOTAL))
    return flat.reshape(1, SEQ, DIM)


# speedup vs baseline: 1.1383x; 1.1383x over previous
import functools

import jax
import jax.numpy as jnp
from jax import lax
from jax.experimental import pallas as pl
from jax.experimental.pallas import tpu as pltpu
from jax.experimental.pallas import tpu_sc as plsc

DIM = 2048
SEQ = 8192
SCALE = DIM ** (-0.5)

NC = 2
NS = 16
NW = NC * NS

TOTAL = SEQ * DIM
PER_W = TOTAL // NW
CHUNK = 32_768
NCHUNK = PER_W // CHUNK
LANES = 16

_mesh = plsc.VectorSubcoreMesh(core_axis_name="c", subcore_axis_name="s")


@functools.partial(
    pl.kernel,
    out_type=jax.ShapeDtypeStruct((TOTAL,), jnp.float32),
    mesh=_mesh,
    scratch_types=[
        pltpu.VMEM((CHUNK,), jnp.float32),
        pltpu.VMEM((CHUNK,), jnp.float32),
        pltpu.SemaphoreType.DMA,
        pltpu.SemaphoreType.DMA,
        pltpu.SemaphoreType.DMA,
        pltpu.SemaphoreType.DMA,
    ],
)
def _scale_copy_sc(emb_hbm, out_hbm, buf0, buf1, gi0, gi1, so0, so1):
    wid = lax.axis_index("s") * NC + lax.axis_index("c")
    base = pl.multiple_of(wid * PER_W, CHUNK)

    bufs = (buf0, buf1)
    gsem = (gi0, gi1)
    ssem = (so0, so1)

    def gather(k, s):
        return pltpu.async_copy(
            emb_hbm.at[pl.ds(base + k * CHUNK, CHUNK)], bufs[s], gsem[s])

    def scatter(k, s):
        return pltpu.async_copy(
            bufs[s], out_hbm.at[pl.ds(base + k * CHUNK, CHUNK)], ssem[s])

    def compute(buf):
        @plsc.parallel_loop(0, CHUNK, step=LANES, unroll=8)
        def _(i):
            buf[pl.ds(i, LANES)] = buf[pl.ds(i, LANES)] * SCALE

    g = [None, None]
    sc = [None, None]
    g[0] = gather(0, 0)
    for k in range(NCHUNK):
        s = k & 1
        o = 1 - s
        if k + 1 < NCHUNK:
            if sc[o] is not None:
                sc[o].wait()
            g[o] = gather(k + 1, o)
        g[s].wait()
        compute(bufs[s])
        sc[s] = scatter(k, s)
    sc[0].wait()
    sc[1].wait()


def kernel(x, emb):
    del x
    flat = _scale_copy_sc(emb.reshape(TOTAL))
    return flat.reshape(1, SEQ, DIM)


# speedup vs baseline: 3.0542x; 2.6831x over previous
import functools

import jax
import jax.numpy as jnp
from jax import lax
from jax.experimental import pallas as pl
from jax.experimental.pallas import tpu as pltpu
from jax.experimental.pallas import tpu_sc as plsc

DIM = 2048
SEQ = 8192
SCALE = DIM ** (-0.5)

NC = 2
NS = 16
NW = NC * NS

ROWS_W = SEQ // NW
CROWS = 8
NCHUNK = ROWS_W // CROWS
LANES = 16
VECS = CROWS * DIM // LANES

_mesh = plsc.VectorSubcoreMesh(core_axis_name="c", subcore_axis_name="s")


@functools.partial(
    pl.kernel,
    out_type=jax.ShapeDtypeStruct((SEQ, DIM), jnp.float32),
    mesh=_mesh,
    scratch_types=[
        pltpu.VMEM((CROWS, DIM), jnp.float32),
        pltpu.VMEM((CROWS, DIM), jnp.float32),
        pltpu.VMEM((CROWS, DIM), jnp.float32),
        pltpu.VMEM((CROWS, DIM), jnp.float32),
        pltpu.SemaphoreType.DMA,
        pltpu.SemaphoreType.DMA,
        pltpu.SemaphoreType.DMA,
        pltpu.SemaphoreType.DMA,
    ],
)
def _scale_copy_sc(emb_hbm, out_hbm, a0, a1, b0, b1, ga0, ga1, sb0, sb1):
    wid = lax.axis_index("s") * NC + lax.axis_index("c")
    row0 = pl.multiple_of(wid * ROWS_W, CROWS)

    abuf = (a0, a1)
    bbuf = (b0, b1)
    gsem = (ga0, ga1)
    ssem = (sb0, sb1)

    def gather(k, s):
        return pltpu.async_copy(
            emb_hbm.at[pl.ds(row0 + k * CROWS, CROWS), :], abuf[s], gsem[s])

    def scatter(k, s):
        return pltpu.async_copy(
            bbuf[s], out_hbm.at[pl.ds(row0 + k * CROWS, CROWS), :], ssem[s])

    def compute(src, dst):
        @plsc.parallel_loop(0, VECS, step=1, unroll=8)
        def _(i):
            r = i >> 7
            c = (i & 127) * LANES
            dst[r, pl.ds(c, LANES)] = src[r, pl.ds(c, LANES)] * SCALE

    g = [None, None]
    sc = [None, None]
    g[0] = gather(0, 0)
    g[1] = gather(1, 1)
    for k in range(NCHUNK):
        s = k & 1
        g[s].wait()
        if sc[s] is not None:
            sc[s].wait()
        compute(abuf[s], bbuf[s])
        if k + 2 < NCHUNK:
            g[s] = gather(k + 2, s)
        sc[s] = scatter(k, s)
    sc[0].wait()
    sc[1].wait()


def kernel(x, emb):
    del x
    return _scale_copy_sc(emb)[None, :, :]


# speedup vs baseline: 3.1491x; 1.0311x over previous
import functools

import jax
import jax.numpy as jnp
from jax import lax
from jax.experimental import pallas as pl
from jax.experimental.pallas import tpu as pltpu
from jax.experimental.pallas import tpu_sc as plsc

DIM = 2048
SEQ = 8192
SCALE = DIM ** (-0.5)

NC = 2
NS = 16
NW = NC * NS

ROWS_W = SEQ // NW
CROWS = 8
NCHUNK = ROWS_W // CROWS
LANES = 16
VECS = CROWS * DIM // LANES

_mesh = plsc.VectorSubcoreMesh(core_axis_name="c", subcore_axis_name="s")


@functools.partial(
    pl.kernel,
    out_type=jax.ShapeDtypeStruct((SEQ, DIM), jnp.float32),
    mesh=_mesh,
    scratch_types=[
        pltpu.VMEM((CROWS, DIM), jnp.float32),
        pltpu.VMEM((CROWS, DIM), jnp.float32),
        pltpu.VMEM((CROWS, DIM), jnp.float32),
        pltpu.VMEM((CROWS, DIM), jnp.float32),
        pltpu.VMEM((CROWS, DIM), jnp.float32),
        pltpu.VMEM((CROWS, DIM), jnp.float32),
        pltpu.SemaphoreType.DMA,
        pltpu.SemaphoreType.DMA,
        pltpu.SemaphoreType.DMA,
        pltpu.SemaphoreType.DMA,
        pltpu.SemaphoreType.DMA,
        pltpu.SemaphoreType.DMA,
    ],
)
def _scale_copy_sc(emb_hbm, out_hbm, a0, a1, a2, b0, b1, b2,
                   ga0, ga1, ga2, sb0, sb1, sb2):
    wid = lax.axis_index("s") * NC + lax.axis_index("c")
    row0 = pl.multiple_of(wid * ROWS_W, CROWS)

    abuf = (a0, a1, a2)
    bbuf = (b0, b1, b2)
    gsem = (ga0, ga1, ga2)
    ssem = (sb0, sb1, sb2)

    def gather(k, s):
        return pltpu.async_copy(
            emb_hbm.at[pl.ds(row0 + k * CROWS, CROWS), :], abuf[s], gsem[s])

    def scatter(k, s):
        return pltpu.async_copy(
            bbuf[s], out_hbm.at[pl.ds(row0 + k * CROWS, CROWS), :], ssem[s])

    def compute(src, dst):
        @plsc.parallel_loop(0, VECS, step=1, unroll=8)
        def _(i):
            r = i >> 7
            c = (i & 127) * LANES
            dst[r, pl.ds(c, LANES)] = src[r, pl.ds(c, LANES)] * SCALE

    DEPTH = 3
    g = [None] * DEPTH
    sc = [None] * DEPTH
    for s in range(DEPTH):
        g[s] = gather(s, s)
    for k in range(NCHUNK):
        s = k % DEPTH
        g[s].wait()
        if sc[s] is not None:
            sc[s].wait()
        compute(abuf[s], bbuf[s])
        if k + DEPTH < NCHUNK:
            g[s] = gather(k + DEPTH, s)
        sc[s] = scatter(k, s)
    for s in range(DEPTH):
        sc[s].wait()


def kernel(x, emb):
    del x
    return _scale_copy_sc(emb)[None, :, :]


# speedup vs baseline: 3.1527x; 1.0012x over previous
import functools

import jax
import jax.numpy as jnp
from jax import lax
from jax.experimental import pallas as pl
from jax.experimental.pallas import tpu as pltpu
from jax.experimental.pallas import tpu_sc as plsc

DIM = 2048
SEQ = 8192
SCALE = DIM ** (-0.5)

NC = 2
NS = 16
NW = NC * NS

ROWS_W = SEQ // NW
CROWS = 16
NCHUNK = ROWS_W // CROWS
NBUF = 3
LANES = 16
VECS = CROWS * DIM // LANES

_mesh = plsc.VectorSubcoreMesh(core_axis_name="c", subcore_axis_name="s")


@functools.partial(
    pl.kernel,
    out_type=jax.ShapeDtypeStruct((SEQ, DIM), jnp.float32),
    mesh=_mesh,
    scratch_types=[
        pltpu.VMEM((CROWS, DIM), jnp.float32),
        pltpu.VMEM((CROWS, DIM), jnp.float32),
        pltpu.VMEM((CROWS, DIM), jnp.float32),
        pltpu.SemaphoreType.DMA,
        pltpu.SemaphoreType.DMA,
        pltpu.SemaphoreType.DMA,
        pltpu.SemaphoreType.DMA,
        pltpu.SemaphoreType.DMA,
        pltpu.SemaphoreType.DMA,
    ],
)
def _scale_copy_sc(emb_hbm, out_hbm, b0, b1, b2, g0, g1, g2, s0, s1, s2):
    wid = lax.axis_index("s") * NC + lax.axis_index("c")
    row0 = pl.multiple_of(wid * ROWS_W, CROWS)

    bufs = (b0, b1, b2)
    gsem = (g0, g1, g2)
    ssem = (s0, s1, s2)

    def gather(k, s):
        return pltpu.async_copy(
            emb_hbm.at[pl.ds(row0 + k * CROWS, CROWS), :], bufs[s], gsem[s])

    def scatter(k, s):
        return pltpu.async_copy(
            bufs[s], out_hbm.at[pl.ds(row0 + k * CROWS, CROWS), :], ssem[s])

    def compute(buf):
        @plsc.parallel_loop(0, VECS, step=1, unroll=8)
        def _(i):
            r = i >> 7
            c = (i & 127) * LANES
            buf[r, pl.ds(c, LANES)] = buf[r, pl.ds(c, LANES)] * SCALE

    g = [None] * NBUF
    sc = [None] * NBUF
    g[0] = gather(0, 0)
    g[1] = gather(1, 1)
    for k in range(NCHUNK):
        s = k % NBUF
        g[s].wait()
        compute(bufs[s])
        sc[s] = scatter(k, s)
        if k + 2 < NCHUNK:
            s2 = (k + 2) % NBUF
            if sc[s2] is not None:
                sc[s2].wait()
            g[s2] = gather(k + 2, s2)
    for s in range(NBUF):
        if sc[s] is not None:
            sc[s].wait()


def kernel(x, emb):
    del x
    return _scale_copy_sc(emb)[None, :, :]
